# baseline (speedup 1.0000x reference)
import functools

import jax
import jax.numpy as jnp
from jax import lax
from jax.experimental import pallas as pl
from jax.experimental.pallas import tpu as pltpu
from jax.experimental.pallas import tpu_sc as plsc

_NUM_FACTORS = 32
_BATCH = 16384
_NUM_CORES = 2
_NUM_SUBCORES = 16
_NW = _NUM_CORES * _NUM_SUBCORES
_RPW = _BATCH // _NW
_CHUNK = 128
_NCH = _RPW // _CHUNK


def _mf_body(user_id, item_id, P, Q, user_bias, item_bias, out,
             uidx, iidx, pu, qi, bu, bi, outv, sem):
    wid = lax.axis_index("c") * _NUM_SUBCORES + lax.axis_index("s")
    base = wid * _RPW

    for j in range(_NCH):
        off = base + j * _CHUNK
        pltpu.sync_copy(user_id.at[pl.ds(off, _CHUNK)], uidx.at[j])
        pltpu.sync_copy(item_id.at[pl.ds(off, _CHUNK)], iidx.at[j])

    copies = []
    for j in range(_NCH):
        dst = pl.ds(j * _CHUNK, _CHUNK)
        copies.append(pltpu.async_copy(P.at[uidx.at[j]], pu.at[dst], sem))
        copies.append(pltpu.async_copy(Q.at[iidx.at[j]], qi.at[dst], sem))
        copies.append(pltpu.async_copy(user_bias.at[uidx.at[j]], bu.at[dst], sem))
        copies.append(pltpu.async_copy(item_bias.at[iidx.at[j]], bi.at[dst], sem))
    for c in copies:
        c.wait()

    zero16 = jnp.zeros((16,), jnp.int32)
    lane = lax.iota(jnp.int32, 16)

    def group(g, carry):
        r = g * 16 + lane
        acc = plsc.load_gather(bu, [r]) + plsc.load_gather(bi, [r])
        for f in range(_NUM_FACTORS):
            fv = jnp.full((16,), f, jnp.int32)
            acc = acc + plsc.load_gather(pu, [r, fv]) * plsc.load_gather(qi, [r, fv])
        outv[pl.ds(g * 16, 16)] = acc
        return carry

    lax.fori_loop(0, _RPW // 16, group, 0)

    pltpu.sync_copy(outv, out.at[pl.ds(base, _RPW)])


@jax.jit
def _mf(user_id, item_id, P, Q, user_bias, item_bias):
    mesh = plsc.VectorSubcoreMesh(core_axis_name="c", subcore_axis_name="s")
    kern = functools.partial(
        pl.kernel,
        out_type=jax.ShapeDtypeStruct((_BATCH,), jnp.float32),
        mesh=mesh,
        compiler_params=pltpu.CompilerParams(
            needs_layout_passes=False, use_tc_tiling_on_sc=False),
        scratch_types=[
            pltpu.VMEM((_NCH, _CHUNK), jnp.int32),
            pltpu.VMEM((_NCH, _CHUNK), jnp.int32),
            pltpu.VMEM((_RPW, _NUM_FACTORS), jnp.float32),
            pltpu.VMEM((_RPW, _NUM_FACTORS), jnp.float32),
            pltpu.VMEM((_RPW,), jnp.float32),
            pltpu.VMEM((_RPW,), jnp.float32),
            pltpu.VMEM((_RPW,), jnp.float32),
            pltpu.SemaphoreType.DMA,
        ],
    )(_mf_body)
    return kern(user_id, item_id, P, Q, user_bias, item_bias)


def kernel(user_id, item_id, P, Q, user_bias, item_bias):
    return _mf(user_id, item_id, P, Q,
               user_bias.reshape(-1), item_bias.reshape(-1))
